# baseline (speedup 1.0000x reference)
import functools

import jax
import jax.numpy as jnp
from jax import lax
from jax.experimental import pallas as pl
from jax.experimental.pallas import tpu as pltpu
from jax.experimental.pallas import tpu_sc as plsc

BATCH = 16384
N_FIELDS = 26
NC = 2
NS = 16
L = 16
NW = NC * NS
B_PER_W = BATCH // NW
IDX_PER_W = B_PER_W * N_FIELDS
NCHUNK = 4
CH_IDX = IDX_PER_W // NCHUNK
CH_ROWS = B_PER_W // NCHUNK
RG = CH_ROWS // L
NBUF = 3

_mesh = plsc.VectorSubcoreMesh(
    core_axis_name="c", subcore_axis_name="s", num_cores=NC, num_subcores=NS
)


@functools.partial(
    pl.kernel,
    out_type=jax.ShapeDtypeStruct((BATCH,), jnp.float32),
    mesh=_mesh,
    scratch_types=[
        pltpu.VMEM((IDX_PER_W,), jnp.int32),
        [pltpu.VMEM((CH_IDX,), jnp.float32) for _ in range(NBUF)],
        pltpu.VMEM((B_PER_W,), jnp.float32),
        pltpu.VMEM((L,), jnp.float32),
        [pltpu.SemaphoreType.DMA for _ in range(NBUF)],
    ],
    compiler_params=pltpu.CompilerParams(needs_layout_passes=False),
)
def _lr_kernel(idx_hbm, w_hbm, b_hbm, out_hbm, idx_v, vals, out_v, b_v, sems):
    wid = lax.axis_index("s") * NC + lax.axis_index("c")
    pltpu.sync_copy(b_hbm, b_v)
    pltpu.sync_copy(idx_hbm.at[pl.ds(wid * IDX_PER_W, IDX_PER_W)], idx_v)

    def fire(k):
        return pltpu.async_copy(
            w_hbm.at[idx_v.at[pl.ds(k * CH_IDX, CH_IDX)]],
            vals[k % NBUF],
            sems[k % NBUF],
        )

    lane = lax.iota(jnp.int32, L) * N_FIELDS
    bvec = b_v[...]

    descs = {k: fire(k) for k in range(min(NBUF - 1, NCHUNK))}
    for k in range(NCHUNK):
        descs.pop(k).wait()
        if k + NBUF - 1 < NCHUNK:
            descs[k + NBUF - 1] = fire(k + NBUF - 1)
        buf = vals[k % NBUF]
        for g in range(RG):
            accs = [bvec, 0.0, 0.0]
            for f in range(N_FIELDS):
                accs[f % 3] = accs[f % 3] + plsc.load_gather(
                    buf, [lane + (g * L * N_FIELDS + f)]
                )
            acc = (accs[0] + accs[1]) + accs[2]
            y = 1.0 / (1.0 + jnp.exp(-acc))
            out_v[pl.ds((k * RG + g) * L, L)] = y

    pltpu.sync_copy(out_v, out_hbm.at[pl.ds(wid * B_PER_W, B_PER_W)])


def kernel(indices, w, b):
    idx_blocks = indices.reshape(-1).astype(jnp.int32)
    w_flat = w.reshape(-1).astype(jnp.float32)
    b16 = jnp.broadcast_to(b.astype(jnp.float32), (L,))
    return _lr_kernel(idx_blocks, w_flat, b16)


# speedup vs baseline: 1.0558x; 1.0558x over previous
import functools

import jax
import jax.numpy as jnp
from jax import lax
from jax.experimental import pallas as pl
from jax.experimental.pallas import tpu as pltpu
from jax.experimental.pallas import tpu_sc as plsc

BATCH = 16384
N_FIELDS = 26
NC, NS, L = 2, 16, 16
NW = NC * NS
B_PER_W = BATCH // NW
IDX_PER_W = B_PER_W * N_FIELDS

_mesh = plsc.VectorSubcoreMesh(
    core_axis_name="c", subcore_axis_name="s", num_cores=NC, num_subcores=NS
)


@functools.partial(
    pl.kernel,
    out_type=jax.ShapeDtypeStruct((BATCH,), jnp.float32),
    mesh=_mesh,
    scratch_types=[
        pltpu.VMEM((IDX_PER_W,), jnp.int32),
        pltpu.VMEM((IDX_PER_W,), jnp.float32),
        pltpu.SemaphoreType.DMA,
    ],
    compiler_params=pltpu.CompilerParams(needs_layout_passes=False),
)
def _probe(idx_hbm, w_hbm, b_hbm, out_hbm, idx_v, vals_v, sem):
    wid = lax.axis_index("s") * NC + lax.axis_index("c")
    pltpu.sync_copy(idx_hbm.at[pl.ds(wid * IDX_PER_W, IDX_PER_W)], idx_v)
    pltpu.async_copy(w_hbm.at[idx_v], vals_v, sem).wait()
    pltpu.sync_copy(
        vals_v.at[pl.ds(0, B_PER_W)], out_hbm.at[pl.ds(wid * B_PER_W, B_PER_W)]
    )


def kernel(indices, w, b):
    idx_flat = indices.reshape(-1).astype(jnp.int32)
    w_flat = w.reshape(-1).astype(jnp.float32)
    return _probe(idx_flat, w_flat, b)


# speedup vs baseline: 1.8261x; 1.7296x over previous
import functools

import jax
import jax.numpy as jnp
from jax import lax
from jax.experimental import pallas as pl
from jax.experimental.pallas import tpu as pltpu
from jax.experimental.pallas import tpu_sc as plsc

BATCH = 16384
N_FIELDS = 26
NC = 2
NS = 16
L = 16
NW = NC * NS
B_PER_W = BATCH // NW
IDX_PER_W = B_PER_W * N_FIELDS
GROUPS = B_PER_W // L

_mesh = plsc.VectorSubcoreMesh(
    core_axis_name="c", subcore_axis_name="s", num_cores=NC, num_subcores=NS
)


@functools.partial(
    pl.kernel,
    out_type=jax.ShapeDtypeStruct((BATCH,), jnp.float32),
    mesh=_mesh,
    scratch_types=[
        pltpu.VMEM((IDX_PER_W,), jnp.int32),
        pltpu.VMEM((IDX_PER_W,), jnp.float32),
        pltpu.VMEM((B_PER_W,), jnp.float32),
        pltpu.VMEM((L,), jnp.float32),
        pltpu.SemaphoreType.DMA,
    ],
    compiler_params=pltpu.CompilerParams(
        needs_layout_passes=False, use_tc_tiling_on_sc=True
    ),
)
def _lr_kernel(idx_hbm, w_hbm, b_hbm, out_hbm, idx_v, vals_v, out_v, b_v, sem):
    wid = lax.axis_index("s") * NC + lax.axis_index("c")
    pltpu.sync_copy(b_hbm, b_v)
    pltpu.sync_copy(idx_hbm.at[pl.ds(wid * IDX_PER_W, IDX_PER_W)], idx_v)
    pltpu.async_copy(w_hbm.at[0].at[idx_v], vals_v, sem).wait()

    lane = lax.iota(jnp.int32, L) * N_FIELDS
    bvec = b_v[...]

    def body(g, carry):
        row0 = g * (L * N_FIELDS)
        accs = [bvec, 0.0, 0.0]
        for f in range(N_FIELDS):
            accs[f % 3] = accs[f % 3] + plsc.load_gather(
                vals_v, [lane + (row0 + f)]
            )
        acc = (accs[0] + accs[1]) + accs[2]
        y = 1.0 / (1.0 + jnp.exp(-acc))
        out_v[pl.ds(g * L, L)] = y
        return carry

    lax.fori_loop(0, GROUPS, body, 0)
    pltpu.sync_copy(out_v, out_hbm.at[pl.ds(wid * B_PER_W, B_PER_W)])


def kernel(indices, w, b):
    idx_flat = indices.reshape(-1).astype(jnp.int32)
    w_t = w.T.astype(jnp.float32)
    b16 = jnp.broadcast_to(b.astype(jnp.float32), (L,))
    return _lr_kernel(idx_flat, w_t, b16)


# speedup vs baseline: 2.5033x; 1.3708x over previous
import functools

import jax
import jax.numpy as jnp
from jax import lax
from jax.experimental import pallas as pl
from jax.experimental.pallas import tpu as pltpu
from jax.experimental.pallas import tpu_sc as plsc

BATCH = 16384
N_FIELDS = 26
NC = 2
NS = 16
L = 16
NW = NC * NS
B_PER_W = BATCH // NW
IDX_PER_W = B_PER_W * N_FIELDS
GROUPS = B_PER_W // L

_mesh = plsc.VectorSubcoreMesh(
    core_axis_name="c", subcore_axis_name="s", num_cores=NC, num_subcores=NS
)


@functools.partial(
    pl.kernel,
    out_type=jax.ShapeDtypeStruct((BATCH,), jnp.float32),
    mesh=_mesh,
    scratch_types=[
        pltpu.VMEM((IDX_PER_W,), jnp.int32),
        pltpu.VMEM((IDX_PER_W,), jnp.float32),
        pltpu.VMEM((B_PER_W,), jnp.float32),
        pltpu.VMEM((L,), jnp.float32),
        pltpu.SemaphoreType.DMA,
    ],
    compiler_params=pltpu.CompilerParams(
        needs_layout_passes=False, use_tc_tiling_on_sc=True
    ),
)
def _lr_kernel(idx_hbm, w_hbm, b_hbm, out_hbm, idx_v, vals_v, out_v, b_v, sem):
    wid = lax.axis_index("s") * NC + lax.axis_index("c")
    base = wid * B_PER_W
    pltpu.sync_copy(b_hbm, b_v)
    idx_copies = [
        pltpu.async_copy(
            idx_hbm.at[f, pl.ds(base, B_PER_W)],
            idx_v.at[pl.ds(f * B_PER_W, B_PER_W)],
            sem,
        )
        for f in range(N_FIELDS)
    ]
    for c in idx_copies:
        c.wait()
    pltpu.async_copy(w_hbm.at[0].at[idx_v], vals_v, sem).wait()

    bvec = b_v[...]

    def body(g, carry):
        accs = [bvec, 0.0, 0.0]
        for f in range(N_FIELDS):
            accs[f % 3] = accs[f % 3] + vals_v[pl.ds(f * B_PER_W + g * L, L)]
        acc = (accs[0] + accs[1]) + accs[2]
        y = 1.0 / (1.0 + jnp.exp(-acc))
        out_v[pl.ds(g * L, L)] = y
        return carry

    lax.fori_loop(0, GROUPS, body, 0)
    pltpu.sync_copy(out_v, out_hbm.at[pl.ds(base, B_PER_W)])


def kernel(indices, w, b):
    idx_t = indices.T.astype(jnp.int32)
    w_t = w.T.astype(jnp.float32)
    b16 = jnp.broadcast_to(b.astype(jnp.float32), (L,))
    return _lr_kernel(idx_t, w_t, b16)
